# baseline (speedup 1.0000x reference)
import functools

import jax
import jax.numpy as jnp
from jax import lax
from jax.experimental import pallas as pl
from jax.experimental.pallas import tpu as pltpu
from jax.experimental.pallas import tpu_sc as plsc

_NUM_ANCHORS = 3
_NOOBJ_SCALE = 50.0
_GRID = 12

_LOG1P_COEFS = (
    5.62195900721818e-07,
    0.9999574870750696,
    -0.4992065685478763,
    0.32697310001391783,
    -0.2228362583278401,
    0.13076503250360005,
    -0.05262485136716543,
    0.010119082927575069,
)


def _softplus_vec(x):
    m = jnp.maximum(x, 0.0)
    u = jnp.exp(-jnp.abs(x))
    p = jnp.full_like(x, _LOG1P_COEFS[-1])
    for c in reversed(_LOG1P_COEFS[:-1]):
        p = p * u + jnp.float32(c)
    return m + p


def _round_up(x, m):
    return -(-x // m) * m


def _extract_body(blk, x_ref, out_ref):
    x = x_ref[...]
    for a in range(_NUM_ANCHORS):
        out_ref[pl.ds(a * blk, blk)] = x[:, 85 * a + 4]


def _make_extract(rows, C):
    import functools as _ft

    blk = rows // _GRID
    assert blk * _GRID == rows
    oblk = _round_up(_NUM_ANCHORS * blk, 1024)
    return (
        pl.pallas_call(
            _ft.partial(_extract_body, blk),
            grid=(_GRID,),
            in_specs=[pl.BlockSpec((blk, C), lambda i: (i, 0))],
            out_specs=pl.BlockSpec((oblk,), lambda i: (i,)),
            out_shape=jax.ShapeDtypeStruct((_GRID * oblk,), jnp.float32),
        ),
        blk,
        oblk,
    )


def _make_sc_reduce(rows, B, blk, oblk):
    info = plsc.get_sparse_core_info()
    NC, NS, L = info.num_cores, info.num_subcores, info.num_lanes
    NW = NC * NS
    CHUNK = 2704
    Q = blk // CHUNK
    assert Q * CHUNK == blk
    NCHUNKS = _GRID * _NUM_ANCHORS * Q
    MAXK = -(-NCHUNKS // NW)
    VECS = CHUNK // L
    scale = jnp.float32(_NOOBJ_SCALE / B)

    mesh = plsc.VectorSubcoreMesh(core_axis_name="c", subcore_axis_name="s")

    @functools.partial(
        pl.kernel,
        mesh=mesh,
        out_type=jax.ShapeDtypeStruct((NW * L,), jnp.float32),
        scratch_types=[
            pltpu.VMEM((CHUNK,), jnp.float32),
            pltpu.VMEM((L,), jnp.float32),
        ],
    )
    def sc_fn(obj_hbm, out_hbm, buf_v, acc_v):
        wid = lax.axis_index("s") * NC + lax.axis_index("c")
        acc_v[...] = jnp.zeros((L,), jnp.float32)
        for k in range(MAXK):
            p = wid + k * NW

            @pl.when(p < NCHUNKS)
            def _do_chunk():
                g = p // (_NUM_ANCHORS * Q)
                r = p - g * (_NUM_ANCHORS * Q)
                a = r // Q
                off = g * oblk + a * blk + (r - a * Q) * CHUNK
                pltpu.sync_copy(obj_hbm.at[pl.ds(off, CHUNK)], buf_v)

                def body(i, acc):
                    x = buf_v[pl.ds(i * L, L)]
                    return acc + _softplus_vec(x)

                acc = lax.fori_loop(0, VECS, body, jnp.zeros((L,), jnp.float32))
                acc_v[...] = acc_v[...] + acc

        acc_v[...] = acc_v[...] * scale
        pltpu.sync_copy(acc_v, out_hbm.at[pl.ds(wid * L, L)])

    return sc_fn


def kernel(predictions, targets):
    S, B, C, G, _ = predictions.shape
    pt = jnp.transpose(predictions, (0, 3, 4, 1, 2))
    rows = S * G * G * B
    ptr = pt.reshape(rows, C)
    extract, blk, oblk = _make_extract(rows, C)
    obj = extract(ptr)
    partials = _make_sc_reduce(rows, B, blk, oblk)(obj)
    return jnp.sum(partials)


# speedup vs baseline: 2.9045x; 2.9045x over previous
import jax
import jax.numpy as jnp
from jax.experimental import pallas as pl
from jax.experimental.pallas import tpu as pltpu

_NUM_ANCHORS = 3
_NOOBJ_SCALE = 50.0
_GRID = 13


def _body(x_ref, out_ref):
    i = pl.program_id(0)

    @pl.when(i == 0)
    def _init():
        out_ref[0, 0] = jnp.float32(0.0)

    x = x_ref[...]
    t0 = jnp.transpose(x[:, 0:128])
    t1 = jnp.transpose(x[:, 128:192])
    total = jnp.float32(0.0)
    for v in (t0[4, :], t0[89, :], t1[46, :]):
        total += jnp.sum(jax.nn.softplus(v))
    out_ref[0, 0] += total


def kernel(predictions, targets):
    S, B, C, G, _ = predictions.shape
    pt = jnp.transpose(predictions, (0, 3, 4, 1, 2))
    rows = S * G * G * B
    ptr = pt.reshape(rows, C)
    block = rows // _GRID
    assert block * _GRID == rows
    out = pl.pallas_call(
        _body,
        grid=(_GRID,),
        in_specs=[pl.BlockSpec((block, C), lambda i: (i, 0))],
        out_specs=pl.BlockSpec((1, 1), lambda i: (0, 0), memory_space=pltpu.SMEM),
        out_shape=jax.ShapeDtypeStruct((1, 1), jnp.float32),
    )(ptr)
    return out[0, 0] * jnp.float32(_NOOBJ_SCALE / B)
